# baseline (speedup 1.0000x reference)
import functools

import jax
import jax.numpy as jnp
from jax import lax
from jax.experimental import pallas as pl
from jax.experimental.pallas import tpu as pltpu
from jax.experimental.pallas import tpu_sc as plsc

N = 10000
E = 320000
D = 128

NC = 2
NS = 16
NW = NC * NS

EPT = E // NW
CHUNK = 128
NB = 2
NT = NB * (-(-EPT // (CHUNK * NB)))
PAD = NT * CHUNK - EPT
NROWS = N + 8

RPT = 632
LAST = N - (NS - 1) * RPT
DEGW = 16


def _sc_mesh():
    return plsc.VectorSubcoreMesh(core_axis_name="c", subcore_axis_name="s")


def _tile_slice_copy(s, src_at, dst_at):
    base = pl.multiple_of(s * RPT, 8)

    @pl.when(s < NS - 1)
    def _():
        pltpu.sync_copy(src_at(base, RPT), dst_at(base, RPT))

    @pl.when(s == NS - 1)
    def _():
        lbase = (NS - 1) * RPT
        pltpu.sync_copy(src_at(lbase, LAST), dst_at(lbase, LAST))


@functools.partial(
    pl.kernel,
    out_type=(
        jax.ShapeDtypeStruct((N, D), jnp.float32),
        jax.ShapeDtypeStruct((N, D), jnp.float32),
    ),
    mesh=_sc_mesh(),
    scratch_types=[
        pltpu.VMEM((NT, CHUNK), jnp.int32),
        pltpu.VMEM((CHUNK, D), jnp.float32),
        pltpu.VMEM_SHARED((NROWS, D), jnp.float32),
        pltpu.SemaphoreType.DMA,
        pltpu.SemaphoreType.DMA,
    ],
)
def _deg_kernel(dstb_hbm, zeros_hbm, ones_hbm, out0, out1,
                dst_all, ones_v, cnt, sa, sb):
    c = lax.axis_index("c")
    s = lax.axis_index("s")
    wid = c * NS + s
    _tile_slice_copy(s, lambda b, n: zeros_hbm.at[pl.ds(b, n)],
                     lambda b, n: cnt.at[pl.ds(b, n)])
    pltpu.sync_copy(ones_hbm, ones_v)
    pltpu.sync_copy(dstb_hbm.at[wid], dst_all)
    plsc.subcore_barrier()

    pltpu.async_copy(ones_v, cnt.at[dst_all.at[0]], sa, add=True)

    def body(t, carry):
        j0 = 2 * t
        pltpu.async_copy(ones_v, cnt.at[dst_all.at[j0 + 1]], sb, add=True)
        pltpu.make_async_copy(ones_v, cnt.at[dst_all.at[j0]], sa).wait()
        pltpu.async_copy(ones_v, cnt.at[dst_all.at[j0 + 2]], sa, add=True)
        pltpu.make_async_copy(ones_v, cnt.at[dst_all.at[j0 + 1]], sb).wait()
        return carry

    lax.fori_loop(0, (NT - 2) // 2, body, 0)
    pltpu.async_copy(ones_v, cnt.at[dst_all.at[NT - 1]], sb, add=True)
    pltpu.make_async_copy(ones_v, cnt.at[dst_all.at[NT - 2]], sa).wait()
    pltpu.make_async_copy(ones_v, cnt.at[dst_all.at[NT - 1]], sb).wait()
    plsc.subcore_barrier()

    @pl.when(c == 0)
    def _():
        _tile_slice_copy(s, lambda b, n: cnt.at[pl.ds(b, n)],
                         lambda b, n: out0.at[pl.ds(b, n)])

    @pl.when(c == 1)
    def _():
        _tile_slice_copy(s, lambda b, n: cnt.at[pl.ds(b, n)],
                         lambda b, n: out1.at[pl.ds(b, n)])


@functools.partial(
    pl.kernel,
    out_type=(
        jax.ShapeDtypeStruct((N, D), jnp.float32),
        jax.ShapeDtypeStruct((N, D), jnp.float32),
    ),
    mesh=_sc_mesh(),
    scratch_types=[
        pltpu.VMEM((CHUNK,), jnp.int32),
        pltpu.VMEM((CHUNK,), jnp.int32),
        pltpu.VMEM((CHUNK, D), jnp.float32),
        pltpu.VMEM_SHARED((NROWS, D), jnp.float32),
        pltpu.SemaphoreType.DMA,
    ],
)
def _agg_kernel(hs_hbm, srcf_hbm, dstb_hbm, zeros_hbm, out0, out1,
                isa, isb, rows_a, agg, ga):
    c = lax.axis_index("c")
    s = lax.axis_index("s")
    wid = c * NS + s

    @pl.when(c == 0)
    def _():
        _tile_slice_copy(s, lambda b, n: hs_hbm.at[pl.ds(b, n)],
                         lambda b, n: agg.at[pl.ds(b, n)])

    @pl.when(c == 1)
    def _():
        _tile_slice_copy(s, lambda b, n: zeros_hbm.at[pl.ds(b, n)],
                         lambda b, n: agg.at[pl.ds(b, n)])

    plsc.subcore_barrier()

    ebase = wid * (NT * CHUNK)

    def body(j, carry):
        off = pl.multiple_of(ebase + j * CHUNK, 8)
        pltpu.sync_copy(srcf_hbm.at[pl.ds(off, CHUNK)], isa)
        pltpu.sync_copy(dstb_hbm.at[pl.ds(off, CHUNK)], isb)
        pltpu.async_copy(hs_hbm.at[isa], rows_a, ga).wait()
        pltpu.sync_copy(rows_a, agg.at[isb], add=True)
        return carry

    lax.fori_loop(0, NT, body, 0)

    plsc.subcore_barrier()

    @pl.when(c == 0)
    def _():
        _tile_slice_copy(s, lambda b, n: agg.at[pl.ds(b, n)],
                         lambda b, n: out0.at[pl.ds(b, n)])

    @pl.when(c == 1)
    def _():
        _tile_slice_copy(s, lambda b, n: agg.at[pl.ds(b, n)],
                         lambda b, n: out1.at[pl.ds(b, n)])


_BLK = 1000
_GRID = N // _BLK


def _mm_scale_body(x_ref, w_ref, d0_ref, d1_ref, o_ref, dv_ref):
    deg = d0_ref[:, 0:1] + d1_ref[:, 0:1] + 1.0
    dinv = lax.rsqrt(deg)
    h = jnp.dot(x_ref[...], w_ref[...], preferred_element_type=jnp.float32)
    o_ref[...] = h * dinv
    dv_ref[...] = jnp.broadcast_to(dinv, (_BLK, DEGW))


def _finish_mm_body(a0_ref, a1_ref, dv_ref, b_ref, w_ref, o_ref):
    dinv = dv_ref[:, 0:1]
    h = (a0_ref[...] + a1_ref[...]) * dinv + b_ref[...]
    o_ref[...] = jnp.dot(h, w_ref[...], preferred_element_type=jnp.float32) * dinv


def _finish_body(a0_ref, a1_ref, dv_ref, b_ref, o_ref):
    dinv = dv_ref[:, 0:1]
    o_ref[...] = (a0_ref[...] + a1_ref[...]) * dinv + b_ref[...]


_row_spec = pl.BlockSpec((_BLK, D), lambda i: (i, 0))
_dv_spec = pl.BlockSpec((_BLK, DEGW), lambda i: (i, 0))
_w_spec = pl.BlockSpec((D, D), lambda i: (0, 0))
_b_spec = pl.BlockSpec((1, D), lambda i: (0, 0))
_out_t = jax.ShapeDtypeStruct((N, D), jnp.float32)
_dv_t = jax.ShapeDtypeStruct((N, DEGW), jnp.float32)

_mm_scale = pl.pallas_call(
    _mm_scale_body,
    grid=(_GRID,),
    in_specs=[_row_spec, _w_spec, _row_spec, _row_spec],
    out_specs=(_row_spec, _dv_spec),
    out_shape=(_out_t, _dv_t),
)

_finish_mm = pl.pallas_call(
    _finish_mm_body,
    grid=(_GRID,),
    in_specs=[_row_spec, _row_spec, _dv_spec, _b_spec, _w_spec],
    out_specs=_row_spec,
    out_shape=_out_t,
)

_finish = pl.pallas_call(
    _finish_body,
    grid=(_GRID,),
    in_specs=[_row_spec, _row_spec, _dv_spec, _b_spec],
    out_specs=_row_spec,
    out_shape=_out_t,
)


def kernel(x, edge_index, W1, b1, W2, b2):
    ei = edge_index.astype(jnp.int32)
    src = ei[0]
    dst = ei[1]
    b1r = b1.reshape(1, D)
    b2r = b2.reshape(1, D)

    srcf = jnp.concatenate(
        [src.reshape(NW, EPT), jnp.zeros((NW, PAD), jnp.int32)], axis=1
    ).reshape(NW * NT * CHUNK)
    dstf = jnp.concatenate(
        [dst.reshape(NW, EPT), jnp.full((NW, PAD), N, jnp.int32)], axis=1
    ).reshape(NW * NT * CHUNK)
    dstb = dstf.reshape(NW, NT, CHUNK)

    ones_rows = jnp.ones((CHUNK, D), jnp.float32)
    zeros_rows = jnp.zeros((N, D), jnp.float32)

    deg0, deg1 = _deg_kernel(dstb, zeros_rows, ones_rows)

    hs1, dinvb = _mm_scale(x, W1, deg0, deg1)
    a10, a11 = _agg_kernel(hs1, srcf, dstf, zeros_rows)
    hs2 = _finish_mm(a10, a11, dinvb, b1r, W2)
    a20, a21 = _agg_kernel(hs2, srcf, dstf, zeros_rows)
    return _finish(a20, a21, dinvb, b2r)


# speedup vs baseline: 1.7984x; 1.7984x over previous
import functools

import jax
import jax.numpy as jnp
from jax import lax
from jax.experimental import pallas as pl
from jax.experimental.pallas import tpu as pltpu
from jax.experimental.pallas import tpu_sc as plsc

N = 10000
E = 320000
D = 128

NC = 2
NS = 16
NW = NC * NS

EPT = E // NW
CHUNK = 128
NFULL = EPT // CHUNK
TAIL = EPT - NFULL * CHUNK

RPT = 632
LAST = N - (NS - 1) * RPT
DEGW = 16


def _sc_mesh():
    return plsc.VectorSubcoreMesh(core_axis_name="c", subcore_axis_name="s")


def _tile_slice_copy(s, src_at, dst_at):
    base = pl.multiple_of(s * RPT, 8)

    @pl.when(s < NS - 1)
    def _():
        pltpu.sync_copy(src_at(base, RPT), dst_at(base, RPT))

    @pl.when(s == NS - 1)
    def _():
        lbase = (NS - 1) * RPT
        pltpu.sync_copy(src_at(lbase, LAST), dst_at(lbase, LAST))


@functools.partial(
    pl.kernel,
    out_type=(
        jax.ShapeDtypeStruct((N, D), jnp.float32),
        jax.ShapeDtypeStruct((N, D), jnp.float32),
    ),
    mesh=_sc_mesh(),
    scratch_types=[
        pltpu.VMEM((CHUNK,), jnp.int32),
        pltpu.VMEM((TAIL,), jnp.int32),
        pltpu.VMEM((CHUNK, D), jnp.float32),
        pltpu.VMEM_SHARED((N, D), jnp.float32),
    ],
)
def _deg_kernel(dst_hbm, zeros_hbm, ones_hbm, out0, out1,
                idx_d, idx_t, ones_v, cnt):
    c = lax.axis_index("c")
    s = lax.axis_index("s")
    wid = c * NS + s
    _tile_slice_copy(s, lambda b, n: zeros_hbm.at[pl.ds(b, n)],
                     lambda b, n: cnt.at[pl.ds(b, n)])
    pltpu.sync_copy(ones_hbm, ones_v)
    plsc.subcore_barrier()

    ebase = wid * EPT

    def body(j, carry):
        off = pl.multiple_of(ebase + j * CHUNK, 8)
        pltpu.sync_copy(dst_hbm.at[pl.ds(off, CHUNK)], idx_d)
        pltpu.sync_copy(ones_v, cnt.at[idx_d], add=True)
        return carry

    lax.fori_loop(0, NFULL, body, 0)
    toff = pl.multiple_of(ebase + NFULL * CHUNK, 8)
    pltpu.sync_copy(dst_hbm.at[pl.ds(toff, TAIL)], idx_t)
    pltpu.sync_copy(ones_v.at[pl.ds(0, TAIL)], cnt.at[idx_t], add=True)
    plsc.subcore_barrier()

    @pl.when(c == 0)
    def _():
        _tile_slice_copy(s, lambda b, n: cnt.at[pl.ds(b, n)],
                         lambda b, n: out0.at[pl.ds(b, n)])

    @pl.when(c == 1)
    def _():
        _tile_slice_copy(s, lambda b, n: cnt.at[pl.ds(b, n)],
                         lambda b, n: out1.at[pl.ds(b, n)])


@functools.partial(
    pl.kernel,
    out_type=(
        jax.ShapeDtypeStruct((N, D), jnp.float32),
        jax.ShapeDtypeStruct((N, D), jnp.float32),
    ),
    mesh=_sc_mesh(),
    scratch_types=[
        pltpu.VMEM((CHUNK,), jnp.int32),
        pltpu.VMEM((CHUNK,), jnp.int32),
        pltpu.VMEM((TAIL,), jnp.int32),
        pltpu.VMEM((TAIL,), jnp.int32),
        pltpu.VMEM((CHUNK, D), jnp.float32),
        pltpu.VMEM((TAIL, D), jnp.float32),
        pltpu.VMEM_SHARED((N, D), jnp.float32),
        pltpu.SemaphoreType.DMA,
    ],
)
def _agg_kernel(hs_hbm, src_hbm, dst_hbm, zeros_hbm, out0, out1,
                idx_s, idx_d, idx_st, idx_dt, rows, rows_t, agg, sem):
    c = lax.axis_index("c")
    s = lax.axis_index("s")
    wid = c * NS + s

    @pl.when(c == 0)
    def _():
        _tile_slice_copy(s, lambda b, n: hs_hbm.at[pl.ds(b, n)],
                         lambda b, n: agg.at[pl.ds(b, n)])

    @pl.when(c == 1)
    def _():
        _tile_slice_copy(s, lambda b, n: zeros_hbm.at[pl.ds(b, n)],
                         lambda b, n: agg.at[pl.ds(b, n)])

    plsc.subcore_barrier()

    ebase = wid * EPT

    def body(j, carry):
        off = pl.multiple_of(ebase + j * CHUNK, 8)
        pltpu.sync_copy(src_hbm.at[pl.ds(off, CHUNK)], idx_s)
        pltpu.sync_copy(dst_hbm.at[pl.ds(off, CHUNK)], idx_d)
        pltpu.async_copy(hs_hbm.at[idx_s], rows, sem).wait()
        pltpu.sync_copy(rows, agg.at[idx_d], add=True)
        return carry

    lax.fori_loop(0, NFULL, body, 0)

    toff = pl.multiple_of(ebase + NFULL * CHUNK, 8)
    pltpu.sync_copy(src_hbm.at[pl.ds(toff, TAIL)], idx_st)
    pltpu.sync_copy(dst_hbm.at[pl.ds(toff, TAIL)], idx_dt)
    pltpu.async_copy(hs_hbm.at[idx_st], rows_t, sem).wait()
    pltpu.sync_copy(rows_t, agg.at[idx_dt], add=True)

    plsc.subcore_barrier()

    @pl.when(c == 0)
    def _():
        _tile_slice_copy(s, lambda b, n: agg.at[pl.ds(b, n)],
                         lambda b, n: out0.at[pl.ds(b, n)])

    @pl.when(c == 1)
    def _():
        _tile_slice_copy(s, lambda b, n: agg.at[pl.ds(b, n)],
                         lambda b, n: out1.at[pl.ds(b, n)])


_BLK = 1000
_GRID = N // _BLK


def _mm_scale_body(x_ref, w_ref, d0_ref, d1_ref, o_ref, dv_ref):
    deg = d0_ref[:, 0:1] + d1_ref[:, 0:1] + 1.0
    dinv = lax.rsqrt(deg)
    h = jnp.dot(x_ref[...], w_ref[...], preferred_element_type=jnp.float32)
    o_ref[...] = h * dinv
    dv_ref[...] = jnp.broadcast_to(dinv, (_BLK, DEGW))


def _finish_mm_body(a0_ref, a1_ref, dv_ref, b_ref, w_ref, o_ref):
    dinv = dv_ref[:, 0:1]
    h = (a0_ref[...] + a1_ref[...]) * dinv + b_ref[...]
    o_ref[...] = jnp.dot(h, w_ref[...], preferred_element_type=jnp.float32) * dinv


def _finish_body(a0_ref, a1_ref, dv_ref, b_ref, o_ref):
    dinv = dv_ref[:, 0:1]
    o_ref[...] = (a0_ref[...] + a1_ref[...]) * dinv + b_ref[...]


_row_spec = pl.BlockSpec((_BLK, D), lambda i: (i, 0))
_dv_spec = pl.BlockSpec((_BLK, DEGW), lambda i: (i, 0))
_w_spec = pl.BlockSpec((D, D), lambda i: (0, 0))
_b_spec = pl.BlockSpec((1, D), lambda i: (0, 0))
_out_t = jax.ShapeDtypeStruct((N, D), jnp.float32)
_dv_t = jax.ShapeDtypeStruct((N, DEGW), jnp.float32)

_mm_scale = pl.pallas_call(
    _mm_scale_body,
    grid=(_GRID,),
    in_specs=[_row_spec, _w_spec, _row_spec, _row_spec],
    out_specs=(_row_spec, _dv_spec),
    out_shape=(_out_t, _dv_t),
)

_finish_mm = pl.pallas_call(
    _finish_mm_body,
    grid=(_GRID,),
    in_specs=[_row_spec, _row_spec, _dv_spec, _b_spec, _w_spec],
    out_specs=_row_spec,
    out_shape=_out_t,
)

_finish = pl.pallas_call(
    _finish_body,
    grid=(_GRID,),
    in_specs=[_row_spec, _row_spec, _dv_spec, _b_spec],
    out_specs=_row_spec,
    out_shape=_out_t,
)


def kernel(x, edge_index, W1, b1, W2, b2):
    ei = edge_index.astype(jnp.int32)
    src = ei[0]
    dst = ei[1]
    b1r = b1.reshape(1, D)
    b2r = b2.reshape(1, D)

    ones_rows = jnp.ones((CHUNK, D), jnp.float32)
    zeros_rows = jnp.zeros((N, D), jnp.float32)

    deg0, deg1 = _deg_kernel(dst, zeros_rows, ones_rows)

    hs1, dinvb = _mm_scale(x, W1, deg0, deg1)
    a10, a11 = _agg_kernel(hs1, src, dst, zeros_rows)
    hs2 = _finish_mm(a10, a11, dinvb, b1r, W2)
    a20, a21 = _agg_kernel(hs2, src, dst, zeros_rows)
    return _finish(a20, a21, dinvb, b2r)
